# baseline (speedup 1.0000x reference)
import math

import jax
import jax.numpy as jnp
from jax.experimental import pallas as pl

_PATCH = (96, 48, 24, 12, 6, 3)
_TOPK = 4
_DM = 1024
_NEG = -1e30
_HI = jax.lax.Precision.HIGHEST


def _gelu(x):
    return 0.5 * x * (1.0 + jax.lax.erf(x * jnp.float32(1.0 / math.sqrt(2.0))))


def _patch_meta(L):
    out = []
    for p in _PATCH:
        st = max(p // 2, 1)
        pn = L // st + 1
        di = _DM // pn
        out.append((p, st, pn, di, p // 2))
    return out


def _rownorm(x):
    mu = jnp.mean(x, axis=1, keepdims=True)
    xc = x - mu
    var = jnp.mean(xc * xc, axis=1, keepdims=True)
    return xc / jnp.sqrt(var + 1e-5)


def _gate_tail(xs, c_ref, s_ref, w1x_ref, w1f_ref, bg1_ref, w2_ref, bg2_ref):
    re = jnp.dot(xs, c_ref[...], precision=_HI)
    im = jnp.dot(xs, s_ref[...], precision=_HI)
    xf = jnp.sqrt(re * re + im * im)
    pre = (jnp.dot(xs.astype(jnp.bfloat16), w1x_ref[...],
                   preferred_element_type=jnp.float32)
           + jnp.dot(xf.astype(jnp.bfloat16), w1f_ref[...],
                     preferred_element_type=jnp.float32) + bg1_ref[...])
    h = _gelu(pre)
    logits = jnp.dot(h.astype(jnp.bfloat16), w2_ref[...],
                     preferred_element_type=jnp.float32) + bg2_ref[...]
    cols = jax.lax.broadcasted_iota(jnp.int32, logits.shape, 1)
    rank = jnp.zeros(logits.shape, jnp.int32)
    for j in range(logits.shape[1]):
        lj = logits[:, j:j + 1]
        rank += ((lj > logits)
                 | ((lj == logits) & (j < cols))).astype(jnp.int32)
    keep = rank < _TOPK
    masked = jnp.where(keep, logits, _NEG)
    m = jnp.max(masked, axis=1, keepdims=True)
    ex = jnp.where(keep, jnp.exp(logits - m), 0.0)
    return ex / jnp.sum(ex, axis=1, keepdims=True)


def _gate0_body(x_ref, c_ref, s_ref, w1x_ref, w1f_ref, bg1_ref, w2_ref,
                bg2_ref, g_ref, xn_ref):
    xn = _rownorm(x_ref[...])
    xn_ref[...] = xn.astype(jnp.bfloat16)
    g_ref[...] = _gate_tail(xn, c_ref, s_ref, w1x_ref, w1f_ref, bg1_ref,
                            w2_ref, bg2_ref)


def _gate1_body(x_ref, p_ref, c_ref, s_ref, w1x_ref, w1f_ref, bg1_ref,
                w2_ref, bg2_ref, g_ref, xn_ref):
    xn = _rownorm(x_ref[...])
    x2 = jnp.dot(xn, p_ref[...], precision=_HI)
    xn_ref[...] = x2.astype(jnp.bfloat16)
    g_ref[...] = _gate_tail(x2, c_ref, s_ref, w1x_ref, w1f_ref, bg1_ref,
                            w2_ref, bg2_ref)


def _expert_body(x_ref, w1_ref, b1_ref, w2_ref, b2_ref, g_ref, o_ref):
    e = pl.program_id(1)
    x = x_ref[...]
    pre = jnp.dot(x, w1_ref[0], preferred_element_type=jnp.float32)
    h = _gelu(pre + b1_ref[0])
    onehot = (jax.lax.broadcasted_iota(jnp.int32, (8, 1), 0) == e
              ).astype(jnp.float32)
    g = jnp.dot(g_ref[...], onehot)
    hb = (h * g).astype(jnp.bfloat16)
    contrib = (jnp.dot(hb, w2_ref[0], preferred_element_type=jnp.float32)
               + g * b2_ref[0])

    @pl.when(e == 0)
    def _():
        o_ref[...] = contrib

    @pl.when(e != 0)
    def _():
        o_ref[...] = o_ref[...] + contrib


def _dft_mats(L):
    F = L // 2 + 1
    lk = (jnp.arange(L, dtype=jnp.int32)[:, None]
          * jnp.arange(F, dtype=jnp.int32)[None, :]) % L
    ang = lk.astype(jnp.float32) * jnp.float32(2.0 * math.pi / L)
    return jnp.cos(ang), jnp.sin(ang)


def _gate_consts(p, L):
    C, S = _dft_mats(L)
    w1x = p['Wg1'][:L].astype(jnp.bfloat16)
    w1f = p['Wg1'][L:].astype(jnp.bfloat16)
    bg1 = p['bg1'][None, :]
    E = p['Wg2'].shape[1]
    w2 = jnp.pad(p['Wg2'], ((0, 0), (0, 8 - E))).astype(jnp.bfloat16)
    bg2 = jnp.concatenate(
        [p['bg2'], jnp.full((8 - E,), _NEG, jnp.float32)])[None, :]
    return C, S, w1x, w1f, bg1, w2, bg2


def _gates0(xraw, p, L, tile):
    N = xraw.shape[0]
    F = L // 2 + 1
    C, S, w1x, w1f, bg1, w2, bg2 = _gate_consts(p, L)
    fix = lambda t: (0, 0)
    return pl.pallas_call(
        _gate0_body,
        grid=(N // tile,),
        in_specs=[
            pl.BlockSpec((tile, L), lambda t: (t, 0)),
            pl.BlockSpec((L, F), fix),
            pl.BlockSpec((L, F), fix),
            pl.BlockSpec((L, 128), fix),
            pl.BlockSpec((F, 128), fix),
            pl.BlockSpec((1, 128), fix),
            pl.BlockSpec((128, 8), fix),
            pl.BlockSpec((1, 8), fix),
        ],
        out_specs=[pl.BlockSpec((tile, 8), lambda t: (t, 0)),
                   pl.BlockSpec((tile, L), lambda t: (t, 0))],
        out_shape=[jax.ShapeDtypeStruct((N, 8), jnp.float32),
                   jax.ShapeDtypeStruct((N, L), jnp.bfloat16)],
    )(xraw, C, S, w1x, w1f, bg1, w2, bg2)


def _gates1(xraw, p, L, tile):
    N = xraw.shape[0]
    F = L // 2 + 1
    C, S, w1x, w1f, bg1, w2, bg2 = _gate_consts(p, L)
    P = 0.5 * (jnp.arange(2 * L)[:, None] // 2
               == jnp.arange(L)[None, :]).astype(jnp.float32)
    fix = lambda t: (0, 0)
    return pl.pallas_call(
        _gate1_body,
        grid=(N // tile,),
        in_specs=[
            pl.BlockSpec((tile, 2 * L), lambda t: (t, 0)),
            pl.BlockSpec((2 * L, L), fix),
            pl.BlockSpec((L, F), fix),
            pl.BlockSpec((L, F), fix),
            pl.BlockSpec((L, 128), fix),
            pl.BlockSpec((F, 128), fix),
            pl.BlockSpec((1, 128), fix),
            pl.BlockSpec((128, 8), fix),
            pl.BlockSpec((1, 8), fix),
        ],
        out_specs=[pl.BlockSpec((tile, 8), lambda t: (t, 0)),
                   pl.BlockSpec((tile, L), lambda t: (t, 0))],
        out_shape=[jax.ShapeDtypeStruct((N, 8), jnp.float32),
                   jax.ShapeDtypeStruct((N, L), jnp.bfloat16)],
    )(xraw, P, C, S, w1x, w1f, bg1, w2, bg2)


def _band_w1(wff, L, st, pl_, pn, di, padl):
    l = jnp.arange(L, dtype=jnp.int32)[:, None]
    n = jnp.arange(pn, dtype=jnp.int32)[None, :]
    q = l + padl - n * st
    valid = (q >= 0) & (q < pl_)
    w = wff[jnp.clip(q, 0, pl_ - 1)]
    w = jnp.where(valid[..., None], w, 0.0).reshape(L, pn * di)
    return jnp.pad(w, ((0, 0), (0, _DM - pn * di)))


def _experts(xb, p, gates, L, tile):
    N = xb.shape[0]
    meta = _patch_meta(L)
    w1s, b1s, w2s, b2s = [], [], [], []
    for (pl_, st, pn, di, padl), ep in zip(meta, p['experts']):
        w1s.append(_band_w1(ep['Wff'], L, st, pl_, pn, di, padl))
        b1s.append(jnp.pad(jnp.tile(ep['bff'], pn), (0, _DM - pn * di)))
        w2s.append(jnp.pad(ep['Wff1'], ((0, _DM - pn * di), (0, 0))))
        b2s.append(ep['bff1'])
    w1 = jnp.stack(w1s).astype(jnp.bfloat16)
    b1 = jnp.stack(b1s)[:, None, :]
    w2 = jnp.stack(w2s).astype(jnp.bfloat16)
    b2 = jnp.stack(b2s)[:, None, :]
    E = len(meta)
    return pl.pallas_call(
        _expert_body,
        grid=(N // tile, E),
        in_specs=[
            pl.BlockSpec((tile, L), lambda t, e: (t, 0)),
            pl.BlockSpec((1, L, _DM), lambda t, e: (e, 0, 0)),
            pl.BlockSpec((1, 1, _DM), lambda t, e: (e, 0, 0)),
            pl.BlockSpec((1, _DM, _DM), lambda t, e: (e, 0, 0)),
            pl.BlockSpec((1, 1, _DM), lambda t, e: (e, 0, 0)),
            pl.BlockSpec((tile, 8), lambda t, e: (t, 0)),
        ],
        out_specs=pl.BlockSpec((tile, _DM), lambda t, e: (t, 0)),
        out_shape=jax.ShapeDtypeStruct((N, _DM), jnp.float32),
    )(xb, w1, b1, w2, b2, gates)


def kernel(x_enc, params):
    B, L, V = x_enc.shape
    N = B * V
    tile = min(512, N)
    xraw = jnp.transpose(x_enc, (0, 2, 1)).reshape(N, L)
    outs = []
    for si in range(2):
        Ls = L // (si + 1)
        p = params['s' + str(si)]
        if si == 0:
            g, xnb = _gates0(xraw, p, Ls, tile)
        else:
            g, xnb = _gates1(xraw, p, Ls, tile)
        o = _experts(xnb, p, g, Ls, tile)
        outs.append(o.reshape(B, V, _DM))
    return jnp.stack(outs, axis=0)


# speedup vs baseline: 2.4655x; 2.4655x over previous
import math

import jax
import jax.numpy as jnp
from jax.experimental import pallas as pl

_PATCH = (96, 48, 24, 12, 6, 3)
_TOPK = 4
_DM = 1024
_NEG = -1e30
_HI = jax.lax.Precision.HIGHEST


def _gelu(x):
    return 0.5 * x * (1.0 + jax.lax.erf(x * jnp.float32(1.0 / math.sqrt(2.0))))


def _patch_meta(L):
    out = []
    for p in _PATCH:
        st = max(p // 2, 1)
        pn = L // st + 1
        di = _DM // pn
        out.append((p, st, pn, di, p // 2))
    return out


def _rownorm(x):
    mu = jnp.mean(x, axis=1, keepdims=True)
    xc = x - mu
    var = jnp.mean(xc * xc, axis=1, keepdims=True)
    return xc / jnp.sqrt(var + 1e-5)


def _gate_tail(xs, c_ref, s_ref, w1x_ref, w1f_ref, bg1_ref, w2_ref, bg2_ref):
    re = jnp.dot(xs, c_ref[...], precision=_HI)
    im = jnp.dot(xs, s_ref[...], precision=_HI)
    xf = jnp.sqrt(re * re + im * im)
    pre = (jnp.dot(xs.astype(jnp.bfloat16), w1x_ref[...],
                   preferred_element_type=jnp.float32)
           + jnp.dot(xf.astype(jnp.bfloat16), w1f_ref[...],
                     preferred_element_type=jnp.float32) + bg1_ref[...])
    h = _gelu(pre)
    logits = jnp.dot(h.astype(jnp.bfloat16), w2_ref[...],
                     preferred_element_type=jnp.float32) + bg2_ref[...]
    cols = jax.lax.broadcasted_iota(jnp.int32, logits.shape, 1)
    rank = jnp.zeros(logits.shape, jnp.int32)
    for j in range(logits.shape[1]):
        lj = logits[:, j:j + 1]
        rank += ((lj > logits)
                 | ((lj == logits) & (j < cols))).astype(jnp.int32)
    keep = rank < _TOPK
    masked = jnp.where(keep, logits, _NEG)
    m = jnp.max(masked, axis=1, keepdims=True)
    ex = jnp.where(keep, jnp.exp(logits - m), 0.0)
    return ex / jnp.sum(ex, axis=1, keepdims=True)


def _gate0_body(x_ref, c_ref, s_ref, w1x_ref, w1f_ref, bg1_ref, w2_ref,
                bg2_ref, g_ref, xn_ref):
    xn = _rownorm(x_ref[...])
    xn_ref[...] = xn.astype(jnp.bfloat16)
    g_ref[...] = _gate_tail(xn, c_ref, s_ref, w1x_ref, w1f_ref, bg1_ref,
                            w2_ref, bg2_ref)


def _gate1_body(x_ref, p_ref, c_ref, s_ref, w1x_ref, w1f_ref, bg1_ref,
                w2_ref, bg2_ref, g_ref, xn_ref):
    xn = _rownorm(x_ref[...])
    x2 = jnp.dot(xn, p_ref[...], precision=_HI)
    xn_ref[...] = x2.astype(jnp.bfloat16)
    g_ref[...] = _gate_tail(x2, c_ref, s_ref, w1x_ref, w1f_ref, bg1_ref,
                            w2_ref, bg2_ref)


def _expert_body(x_ref, w1_ref, b1_ref, w2_ref, b2_ref, g_ref, o_ref):
    e = pl.program_id(1)
    x = x_ref[...]
    pre = jnp.dot(x, w1_ref[0], preferred_element_type=jnp.float32)
    h = _gelu(pre + b1_ref[0])
    onehot = (jax.lax.broadcasted_iota(jnp.int32, (8, 1), 0) == e
              ).astype(jnp.float32)
    g = jnp.dot(g_ref[...], onehot)
    hb = (h * g).astype(jnp.bfloat16)
    contrib = (jnp.dot(hb, w2_ref[0], preferred_element_type=jnp.float32)
               + g * b2_ref[0])

    @pl.when(e == 0)
    def _():
        o_ref[...] = contrib

    @pl.when(e != 0)
    def _():
        o_ref[...] = o_ref[...] + contrib


def _dft_mats(L):
    F = L // 2 + 1
    lk = (jnp.arange(L, dtype=jnp.int32)[:, None]
          * jnp.arange(F, dtype=jnp.int32)[None, :]) % L
    ang = lk.astype(jnp.float32) * jnp.float32(2.0 * math.pi / L)
    return jnp.cos(ang), jnp.sin(ang)


def _gate_consts(p, L):
    C, S = _dft_mats(L)
    w1x = p['Wg1'][:L].astype(jnp.bfloat16)
    w1f = p['Wg1'][L:].astype(jnp.bfloat16)
    bg1 = p['bg1'][None, :]
    E = p['Wg2'].shape[1]
    w2 = jnp.pad(p['Wg2'], ((0, 0), (0, 8 - E))).astype(jnp.bfloat16)
    bg2 = jnp.concatenate(
        [p['bg2'], jnp.full((8 - E,), _NEG, jnp.float32)])[None, :]
    return C, S, w1x, w1f, bg1, w2, bg2


def _gates0(xraw, p, L, tile):
    N = xraw.shape[0]
    F = L // 2 + 1
    C, S, w1x, w1f, bg1, w2, bg2 = _gate_consts(p, L)
    fix = lambda t: (0, 0)
    return pl.pallas_call(
        _gate0_body,
        grid=(N // tile,),
        in_specs=[
            pl.BlockSpec((tile, L), lambda t: (t, 0)),
            pl.BlockSpec((L, F), fix),
            pl.BlockSpec((L, F), fix),
            pl.BlockSpec((L, 128), fix),
            pl.BlockSpec((F, 128), fix),
            pl.BlockSpec((1, 128), fix),
            pl.BlockSpec((128, 8), fix),
            pl.BlockSpec((1, 8), fix),
        ],
        out_specs=[pl.BlockSpec((tile, 8), lambda t: (t, 0)),
                   pl.BlockSpec((tile, L), lambda t: (t, 0))],
        out_shape=[jax.ShapeDtypeStruct((N, 8), jnp.float32),
                   jax.ShapeDtypeStruct((N, L), jnp.bfloat16)],
    )(xraw, C, S, w1x, w1f, bg1, w2, bg2)


def _gates1(xraw, p, L, tile):
    N = xraw.shape[0]
    F = L // 2 + 1
    C, S, w1x, w1f, bg1, w2, bg2 = _gate_consts(p, L)
    P = 0.5 * (jnp.arange(2 * L)[:, None] // 2
               == jnp.arange(L)[None, :]).astype(jnp.float32)
    fix = lambda t: (0, 0)
    return pl.pallas_call(
        _gate1_body,
        grid=(N // tile,),
        in_specs=[
            pl.BlockSpec((tile, 2 * L), lambda t: (t, 0)),
            pl.BlockSpec((2 * L, L), fix),
            pl.BlockSpec((L, F), fix),
            pl.BlockSpec((L, F), fix),
            pl.BlockSpec((L, 128), fix),
            pl.BlockSpec((F, 128), fix),
            pl.BlockSpec((1, 128), fix),
            pl.BlockSpec((128, 8), fix),
            pl.BlockSpec((1, 8), fix),
        ],
        out_specs=[pl.BlockSpec((tile, 8), lambda t: (t, 0)),
                   pl.BlockSpec((tile, L), lambda t: (t, 0))],
        out_shape=[jax.ShapeDtypeStruct((N, 8), jnp.float32),
                   jax.ShapeDtypeStruct((N, L), jnp.bfloat16)],
    )(xraw, P, C, S, w1x, w1f, bg1, w2, bg2)


def _band_w1(wff, L, st, pl_, pn, di, padl):
    l = jnp.arange(L, dtype=jnp.int32)[:, None, None]
    n = jnp.arange(pn, dtype=jnp.int32)[None, :, None]
    q = l + padl - n * st
    onehot = (q == jnp.arange(pl_, dtype=jnp.int32)[None, None, :]
              ).astype(jnp.bfloat16).reshape(L * pn, pl_)
    w = jnp.dot(onehot, wff.astype(jnp.bfloat16),
                preferred_element_type=jnp.bfloat16).reshape(L, pn * di)
    return jnp.pad(w, ((0, 0), (0, _DM - pn * di)))


def _experts(xb, p, gates, L, tile):
    N = xb.shape[0]
    meta = _patch_meta(L)
    w1s, b1s, w2s, b2s = [], [], [], []
    for (pl_, st, pn, di, padl), ep in zip(meta, p['experts']):
        w1s.append(_band_w1(ep['Wff'], L, st, pl_, pn, di, padl))
        b1s.append(jnp.pad(jnp.tile(ep['bff'], pn), (0, _DM - pn * di)))
        w2s.append(jnp.pad(ep['Wff1'].astype(jnp.bfloat16),
                           ((0, _DM - pn * di), (0, 0))))
        b2s.append(ep['bff1'])
    w1 = jnp.stack(w1s)
    b1 = jnp.stack(b1s)[:, None, :]
    w2 = jnp.stack(w2s)
    b2 = jnp.stack(b2s)[:, None, :]
    E = len(meta)
    return pl.pallas_call(
        _expert_body,
        grid=(N // tile, E),
        in_specs=[
            pl.BlockSpec((tile, L), lambda t, e: (t, 0)),
            pl.BlockSpec((1, L, _DM), lambda t, e: (e, 0, 0)),
            pl.BlockSpec((1, 1, _DM), lambda t, e: (e, 0, 0)),
            pl.BlockSpec((1, _DM, _DM), lambda t, e: (e, 0, 0)),
            pl.BlockSpec((1, 1, _DM), lambda t, e: (e, 0, 0)),
            pl.BlockSpec((tile, 8), lambda t, e: (t, 0)),
        ],
        out_specs=pl.BlockSpec((tile, _DM), lambda t, e: (t, 0)),
        out_shape=jax.ShapeDtypeStruct((N, _DM), jnp.float32),
    )(xb, w1, b1, w2, b2, gates)


def kernel(x_enc, params):
    B, L, V = x_enc.shape
    N = B * V
    tile = min(512, N)
    xraw = jnp.transpose(x_enc, (0, 2, 1)).reshape(N, L)
    outs = []
    for si in range(2):
        Ls = L // (si + 1)
        p = params['s' + str(si)]
        if si == 0:
            g, xnb = _gates0(xraw, p, Ls, tile)
        else:
            g, xnb = _gates1(xraw, p, Ls, tile)
        o = _experts(xnb, p, g, Ls, tile)
        outs.append(o.reshape(B, V, _DM))
    return jnp.stack(outs, axis=0)


# speedup vs baseline: 3.2126x; 1.3030x over previous
import math

import jax
import jax.numpy as jnp
from jax.experimental import pallas as pl

_PATCH = (96, 48, 24, 12, 6, 3)
_TOPK = 4
_DM = 1024
_NEG = -1e30
_HI = jax.lax.Precision.HIGHEST


def _gelu(x):
    return 0.5 * x * (1.0 + jax.lax.erf(x * jnp.float32(1.0 / math.sqrt(2.0))))


def _patch_meta(L):
    out = []
    for p in _PATCH:
        st = max(p // 2, 1)
        pn = L // st + 1
        di = _DM // pn
        out.append((p, st, pn, di, p // 2))
    return out


def _rownorm(x):
    mu = jnp.mean(x, axis=1, keepdims=True)
    xc = x - mu
    var = jnp.mean(xc * xc, axis=1, keepdims=True)
    return xc / jnp.sqrt(var + 1e-5)


def _gate_tail(xs, c_ref, s_ref, w1x_ref, w1f_ref, bg1_ref, w2_ref, bg2_ref):
    re = jnp.dot(xs, c_ref[...], precision=_HI)
    im = jnp.dot(xs, s_ref[...], precision=_HI)
    xf = jnp.sqrt(re * re + im * im)
    pre = (jnp.dot(xs.astype(jnp.bfloat16), w1x_ref[...],
                   preferred_element_type=jnp.float32)
           + jnp.dot(xf.astype(jnp.bfloat16), w1f_ref[...],
                     preferred_element_type=jnp.float32) + bg1_ref[...])
    h = _gelu(pre)
    logits = jnp.dot(h.astype(jnp.bfloat16), w2_ref[...],
                     preferred_element_type=jnp.float32) + bg2_ref[...]
    cols = jax.lax.broadcasted_iota(jnp.int32, logits.shape, 1)
    rank = jnp.zeros(logits.shape, jnp.int32)
    for j in range(logits.shape[1]):
        lj = logits[:, j:j + 1]
        rank += ((lj > logits)
                 | ((lj == logits) & (j < cols))).astype(jnp.int32)
    keep = rank < _TOPK
    masked = jnp.where(keep, logits, _NEG)
    m = jnp.max(masked, axis=1, keepdims=True)
    ex = jnp.where(keep, jnp.exp(logits - m), 0.0)
    return ex / jnp.sum(ex, axis=1, keepdims=True)


def _gate0_body(x_ref, c_ref, s_ref, w1x_ref, w1f_ref, bg1_ref, w2_ref,
                bg2_ref, g_ref, xn_ref):
    xn = _rownorm(x_ref[...])
    xn_ref[...] = xn.astype(jnp.bfloat16)
    g_ref[...] = _gate_tail(xn, c_ref, s_ref, w1x_ref, w1f_ref, bg1_ref,
                            w2_ref, bg2_ref)


def _gate1_body(x_ref, p_ref, c_ref, s_ref, w1x_ref, w1f_ref, bg1_ref,
                w2_ref, bg2_ref, g_ref, xn_ref):
    xn = _rownorm(x_ref[...])
    x2 = jnp.dot(xn, p_ref[...], precision=_HI)
    xn_ref[...] = x2.astype(jnp.bfloat16)
    g_ref[...] = _gate_tail(x2, c_ref, s_ref, w1x_ref, w1f_ref, bg1_ref,
                            w2_ref, bg2_ref)


def _expert_body(x_ref, w1_ref, b1_ref, w2_ref, b2_ref, g_ref, o_ref):
    e = pl.program_id(1)
    x = x_ref[...]
    pre = jnp.dot(x, w1_ref[0], preferred_element_type=jnp.float32)
    h = _gelu(pre + b1_ref[0])
    onehot = (jax.lax.broadcasted_iota(jnp.int32, (8, 1), 0) == e
              ).astype(jnp.float32)
    g = jnp.dot(g_ref[...], onehot)
    hb = (h * g).astype(jnp.bfloat16)
    contrib = (jnp.dot(hb, w2_ref[0], preferred_element_type=jnp.float32)
               + g * b2_ref[0])

    @pl.when(e == 0)
    def _():
        o_ref[...] = contrib

    @pl.when(e != 0)
    def _():
        o_ref[...] = o_ref[...] + contrib


def _dft_mats(L):
    F = L // 2 + 1
    lk = (jnp.arange(L, dtype=jnp.int32)[:, None]
          * jnp.arange(F, dtype=jnp.int32)[None, :]) % L
    ang = lk.astype(jnp.float32) * jnp.float32(2.0 * math.pi / L)
    return jnp.cos(ang), jnp.sin(ang)


def _gate_consts(p, L):
    C, S = _dft_mats(L)
    w1x = p['Wg1'][:L].astype(jnp.bfloat16)
    w1f = p['Wg1'][L:].astype(jnp.bfloat16)
    bg1 = p['bg1'][None, :]
    E = p['Wg2'].shape[1]
    w2 = jnp.pad(p['Wg2'], ((0, 0), (0, 8 - E))).astype(jnp.bfloat16)
    bg2 = jnp.concatenate(
        [p['bg2'], jnp.full((8 - E,), _NEG, jnp.float32)])[None, :]
    return C, S, w1x, w1f, bg1, w2, bg2


def _gates0(xraw, p, L, tile):
    N = xraw.shape[0]
    F = L // 2 + 1
    C, S, w1x, w1f, bg1, w2, bg2 = _gate_consts(p, L)
    fix = lambda t: (0, 0)
    return pl.pallas_call(
        _gate0_body,
        grid=(N // tile,),
        in_specs=[
            pl.BlockSpec((tile, L), lambda t: (t, 0)),
            pl.BlockSpec((L, F), fix),
            pl.BlockSpec((L, F), fix),
            pl.BlockSpec((L, 128), fix),
            pl.BlockSpec((F, 128), fix),
            pl.BlockSpec((1, 128), fix),
            pl.BlockSpec((128, 8), fix),
            pl.BlockSpec((1, 8), fix),
        ],
        out_specs=[pl.BlockSpec((tile, 8), lambda t: (t, 0)),
                   pl.BlockSpec((tile, L), lambda t: (t, 0))],
        out_shape=[jax.ShapeDtypeStruct((N, 8), jnp.float32),
                   jax.ShapeDtypeStruct((N, L), jnp.bfloat16)],
    )(xraw, C, S, w1x, w1f, bg1, w2, bg2)


def _gates1(xraw, p, L, tile):
    N = xraw.shape[0]
    F = L // 2 + 1
    C, S, w1x, w1f, bg1, w2, bg2 = _gate_consts(p, L)
    P = 0.5 * (jnp.arange(2 * L)[:, None] // 2
               == jnp.arange(L)[None, :]).astype(jnp.float32)
    fix = lambda t: (0, 0)
    return pl.pallas_call(
        _gate1_body,
        grid=(N // tile,),
        in_specs=[
            pl.BlockSpec((tile, 2 * L), lambda t: (t, 0)),
            pl.BlockSpec((2 * L, L), fix),
            pl.BlockSpec((L, F), fix),
            pl.BlockSpec((L, F), fix),
            pl.BlockSpec((L, 128), fix),
            pl.BlockSpec((F, 128), fix),
            pl.BlockSpec((1, 128), fix),
            pl.BlockSpec((128, 8), fix),
            pl.BlockSpec((1, 8), fix),
        ],
        out_specs=[pl.BlockSpec((tile, 8), lambda t: (t, 0)),
                   pl.BlockSpec((tile, L), lambda t: (t, 0))],
        out_shape=[jax.ShapeDtypeStruct((N, 8), jnp.float32),
                   jax.ShapeDtypeStruct((N, L), jnp.bfloat16)],
    )(xraw, P, C, S, w1x, w1f, bg1, w2, bg2)


def _band_w1(wff, L, st, pl_, pn, di, padl):
    R = L + pl_
    P = R + st
    buf = jnp.concatenate(
        [wff.astype(jnp.bfloat16),
         jnp.zeros((P - pl_, di), jnp.bfloat16)], axis=0)
    flat = jnp.tile(buf, (pn, 1))[:pn * R]
    t = flat.reshape(pn, R, di)[:, padl:padl + L]
    w = jnp.transpose(t, (1, 0, 2)).reshape(L, pn * di)
    return jnp.pad(w, ((0, 0), (0, _DM - pn * di)))


def _experts(xb, p, gates, L, tile):
    N = xb.shape[0]
    meta = _patch_meta(L)
    w1s, b1s, w2s, b2s = [], [], [], []
    for (pl_, st, pn, di, padl), ep in zip(meta, p['experts']):
        w1s.append(_band_w1(ep['Wff'], L, st, pl_, pn, di, padl))
        b1s.append(jnp.pad(jnp.tile(ep['bff'], pn), (0, _DM - pn * di)))
        w2s.append(jnp.pad(ep['Wff1'].astype(jnp.bfloat16),
                           ((0, _DM - pn * di), (0, 0))))
        b2s.append(ep['bff1'])
    w1 = jnp.stack(w1s)
    b1 = jnp.stack(b1s)[:, None, :]
    w2 = jnp.stack(w2s)
    b2 = jnp.stack(b2s)[:, None, :]
    E = len(meta)
    return pl.pallas_call(
        _expert_body,
        grid=(N // tile, E),
        in_specs=[
            pl.BlockSpec((tile, L), lambda t, e: (t, 0)),
            pl.BlockSpec((1, L, _DM), lambda t, e: (e, 0, 0)),
            pl.BlockSpec((1, 1, _DM), lambda t, e: (e, 0, 0)),
            pl.BlockSpec((1, _DM, _DM), lambda t, e: (e, 0, 0)),
            pl.BlockSpec((1, 1, _DM), lambda t, e: (e, 0, 0)),
            pl.BlockSpec((tile, 8), lambda t, e: (t, 0)),
        ],
        out_specs=pl.BlockSpec((tile, _DM), lambda t, e: (t, 0)),
        out_shape=jax.ShapeDtypeStruct((N, _DM), jnp.float32),
    )(xb, w1, b1, w2, b2, gates)


def kernel(x_enc, params):
    B, L, V = x_enc.shape
    N = B * V
    tile = min(512, N)
    xraw = jnp.transpose(x_enc, (0, 2, 1)).reshape(N, L)
    outs = []
    for si in range(2):
        Ls = L // (si + 1)
        p = params['s' + str(si)]
        if si == 0:
            g, xnb = _gates0(xraw, p, Ls, tile)
        else:
            g, xnb = _gates1(xraw, p, Ls, tile)
        o = _experts(xnb, p, g, Ls, tile)
        outs.append(o.reshape(B, V, _DM))
    return jnp.stack(outs, axis=0)


# speedup vs baseline: 3.2731x; 1.0188x over previous
import math

import jax
import jax.numpy as jnp
from jax.experimental import pallas as pl

_PATCH = (96, 48, 24, 12, 6, 3)
_TOPK = 4
_DM = 1024
_NEG = -1e30
_HI = jax.lax.Precision.HIGHEST


def _gelu(x):
    return 0.5 * x * (1.0 + jax.lax.erf(x * jnp.float32(1.0 / math.sqrt(2.0))))


def _patch_meta(L):
    out = []
    for p in _PATCH:
        st = max(p // 2, 1)
        pn = L // st + 1
        di = _DM // pn
        out.append((p, st, pn, di, p // 2))
    return out


def _rownorm(x):
    mu = jnp.mean(x, axis=1, keepdims=True)
    xc = x - mu
    var = jnp.mean(xc * xc, axis=1, keepdims=True)
    return xc / jnp.sqrt(var + 1e-5)


def _gate_tail(xs, c_ref, s_ref, w1x_ref, w1f_ref, bg1_ref, w2_ref, bg2_ref):
    re = jnp.dot(xs, c_ref[...], precision=_HI)
    im = jnp.dot(xs, s_ref[...], precision=_HI)
    xf = jnp.sqrt(re * re + im * im)
    pre = (jnp.dot(xs.astype(jnp.bfloat16), w1x_ref[...],
                   preferred_element_type=jnp.float32)
           + jnp.dot(xf.astype(jnp.bfloat16), w1f_ref[...],
                     preferred_element_type=jnp.float32) + bg1_ref[...])
    h = _gelu(pre)
    logits = jnp.dot(h.astype(jnp.bfloat16), w2_ref[...],
                     preferred_element_type=jnp.float32) + bg2_ref[...]
    cols = jax.lax.broadcasted_iota(jnp.int32, logits.shape, 1)
    rank = jnp.zeros(logits.shape, jnp.int32)
    for j in range(logits.shape[1]):
        lj = logits[:, j:j + 1]
        rank += ((lj > logits)
                 | ((lj == logits) & (j < cols))).astype(jnp.int32)
    keep = rank < _TOPK
    masked = jnp.where(keep, logits, _NEG)
    m = jnp.max(masked, axis=1, keepdims=True)
    ex = jnp.where(keep, jnp.exp(logits - m), 0.0)
    return ex / jnp.sum(ex, axis=1, keepdims=True)


def _gate0_body(x_ref, c_ref, s_ref, w1x_ref, w1f_ref, bg1_ref, w2_ref,
                bg2_ref, g_ref, xn_ref):
    xn = _rownorm(x_ref[...])
    xn_ref[...] = xn.astype(jnp.bfloat16)
    g_ref[...] = _gate_tail(xn, c_ref, s_ref, w1x_ref, w1f_ref, bg1_ref,
                            w2_ref, bg2_ref)


def _gate1_body(x_ref, p_ref, c_ref, s_ref, w1x_ref, w1f_ref, bg1_ref,
                w2_ref, bg2_ref, g_ref, xn_ref):
    xn = _rownorm(x_ref[...])
    x2 = jnp.dot(xn, p_ref[...], precision=_HI)
    xn_ref[...] = x2.astype(jnp.bfloat16)
    g_ref[...] = _gate_tail(x2, c_ref, s_ref, w1x_ref, w1f_ref, bg1_ref,
                            w2_ref, bg2_ref)


def _expert_body(x_ref, w1_ref, b1_ref, w2_ref, b2_ref, g_ref, o_ref):
    e = pl.program_id(1)
    x = x_ref[...]
    pre = jnp.dot(x, w1_ref[0], preferred_element_type=jnp.float32)
    h = _gelu(pre + b1_ref[0])
    onehot = (jax.lax.broadcasted_iota(jnp.int32, (8, 1), 0) == e
              ).astype(jnp.float32)
    g = jnp.dot(g_ref[...], onehot)
    hb = (h * g).astype(jnp.bfloat16)
    contrib = (jnp.dot(hb, w2_ref[0], preferred_element_type=jnp.float32)
               + g * b2_ref[0])

    @pl.when(e == 0)
    def _():
        o_ref[...] = contrib

    @pl.when(e != 0)
    def _():
        o_ref[...] = o_ref[...] + contrib


def _dft_mats(L):
    F = L // 2 + 1
    lk = (jnp.arange(L, dtype=jnp.int32)[:, None]
          * jnp.arange(F, dtype=jnp.int32)[None, :]) % L
    ang = lk.astype(jnp.float32) * jnp.float32(2.0 * math.pi / L)
    return jnp.cos(ang), jnp.sin(ang)


def _gate_consts(p, L):
    C, S = _dft_mats(L)
    w1x = p['Wg1'][:L].astype(jnp.bfloat16)
    w1f = p['Wg1'][L:].astype(jnp.bfloat16)
    bg1 = p['bg1'][None, :]
    E = p['Wg2'].shape[1]
    w2 = jnp.pad(p['Wg2'], ((0, 0), (0, 8 - E))).astype(jnp.bfloat16)
    bg2 = jnp.concatenate(
        [p['bg2'], jnp.full((8 - E,), _NEG, jnp.float32)])[None, :]
    return C, S, w1x, w1f, bg1, w2, bg2


def _gates0(xraw, p, L, tile):
    N = xraw.shape[0]
    F = L // 2 + 1
    C, S, w1x, w1f, bg1, w2, bg2 = _gate_consts(p, L)
    fix = lambda t: (0, 0)
    return pl.pallas_call(
        _gate0_body,
        grid=(N // tile,),
        in_specs=[
            pl.BlockSpec((tile, L), lambda t: (t, 0)),
            pl.BlockSpec((L, F), fix),
            pl.BlockSpec((L, F), fix),
            pl.BlockSpec((L, 128), fix),
            pl.BlockSpec((F, 128), fix),
            pl.BlockSpec((1, 128), fix),
            pl.BlockSpec((128, 8), fix),
            pl.BlockSpec((1, 8), fix),
        ],
        out_specs=[pl.BlockSpec((tile, 8), lambda t: (t, 0)),
                   pl.BlockSpec((tile, L), lambda t: (t, 0))],
        out_shape=[jax.ShapeDtypeStruct((N, 8), jnp.float32),
                   jax.ShapeDtypeStruct((N, L), jnp.bfloat16)],
    )(xraw, C, S, w1x, w1f, bg1, w2, bg2)


def _gates1(xraw, p, L, tile):
    N = xraw.shape[0]
    F = L // 2 + 1
    C, S, w1x, w1f, bg1, w2, bg2 = _gate_consts(p, L)
    P = 0.5 * (jnp.arange(2 * L)[:, None] // 2
               == jnp.arange(L)[None, :]).astype(jnp.float32)
    fix = lambda t: (0, 0)
    return pl.pallas_call(
        _gate1_body,
        grid=(N // tile,),
        in_specs=[
            pl.BlockSpec((tile, 2 * L), lambda t: (t, 0)),
            pl.BlockSpec((2 * L, L), fix),
            pl.BlockSpec((L, F), fix),
            pl.BlockSpec((L, F), fix),
            pl.BlockSpec((L, 128), fix),
            pl.BlockSpec((F, 128), fix),
            pl.BlockSpec((1, 128), fix),
            pl.BlockSpec((128, 8), fix),
            pl.BlockSpec((1, 8), fix),
        ],
        out_specs=[pl.BlockSpec((tile, 8), lambda t: (t, 0)),
                   pl.BlockSpec((tile, L), lambda t: (t, 0))],
        out_shape=[jax.ShapeDtypeStruct((N, 8), jnp.float32),
                   jax.ShapeDtypeStruct((N, L), jnp.bfloat16)],
    )(xraw, P, C, S, w1x, w1f, bg1, w2, bg2)


def _band_w1(wff, L, st, pl_, pn, di, padl):
    R = L + pl_
    P = R + st
    buf = jnp.concatenate(
        [wff.astype(jnp.bfloat16),
         jnp.zeros((P - pl_, di), jnp.bfloat16)], axis=0)
    flat = jnp.tile(buf, (pn, 1))[:pn * R]
    t = flat.reshape(pn, R, di)[:, padl:padl + L]
    w = jnp.transpose(t, (1, 0, 2)).reshape(L, pn * di)
    return jnp.pad(w, ((0, 0), (0, _DM - pn * di)))


def _experts(xb, p, gates, L, tile):
    N = xb.shape[0]
    meta = _patch_meta(L)
    w1s, b1s, w2s, b2s = [], [], [], []
    for (pl_, st, pn, di, padl), ep in zip(meta, p['experts']):
        w1s.append(_band_w1(ep['Wff'], L, st, pl_, pn, di, padl))
        b1s.append(jnp.pad(jnp.tile(ep['bff'], pn), (0, _DM - pn * di)))
        w2s.append(jnp.pad(ep['Wff1'].astype(jnp.bfloat16),
                           ((0, _DM - pn * di), (0, 0))))
        b2s.append(ep['bff1'])
    w1 = jnp.stack(w1s)
    b1 = jnp.stack(b1s)[:, None, :]
    w2 = jnp.stack(w2s)
    b2 = jnp.stack(b2s)[:, None, :]
    E = len(meta)
    return pl.pallas_call(
        _expert_body,
        grid=(N // tile, E),
        in_specs=[
            pl.BlockSpec((tile, L), lambda t, e: (t, 0)),
            pl.BlockSpec((1, L, _DM), lambda t, e: (e, 0, 0)),
            pl.BlockSpec((1, 1, _DM), lambda t, e: (e, 0, 0)),
            pl.BlockSpec((1, _DM, _DM), lambda t, e: (e, 0, 0)),
            pl.BlockSpec((1, 1, _DM), lambda t, e: (e, 0, 0)),
            pl.BlockSpec((tile, 8), lambda t, e: (t, 0)),
        ],
        out_specs=pl.BlockSpec((tile, _DM), lambda t, e: (t, 0)),
        out_shape=jax.ShapeDtypeStruct((N, _DM), jnp.float32),
    )(xb, w1, b1, w2, b2, gates)


def kernel(x_enc, params):
    B, L, V = x_enc.shape
    N = B * V
    tile = min(512, N)
    xraw = jnp.transpose(x_enc, (0, 2, 1)).reshape(N, L)
    outs = []
    for si in range(2):
        Ls = L // (si + 1)
        p = params['s' + str(si)]
        if si == 0:
            g, xnb = _gates0(xraw, p, Ls, tile)
        else:
            g, xnb = _gates1(xraw, p, Ls, tile)
        o = _experts(xnb, p, g, Ls, N)
        outs.append(o.reshape(B, V, _DM))
    return jnp.stack(outs, axis=0)
